# baseline (speedup 1.0000x reference)
import functools

import jax
import jax.numpy as jnp
from jax import lax
from jax.experimental import pallas as pl
from jax.experimental.pallas import tpu as pltpu
from jax.experimental.pallas import tpu_sc as plsc

BATCH = 16384
SEQ = 200
DIM = 32
LANES = 128
NBT = BATCH // LANES
NC, NS = 2, 16
NW = NC * NS
BT_PER_W = NBT // NW

_mesh = plsc.VectorSubcoreMesh(core_axis_name="c", subcore_axis_name="s")


@functools.partial(
    pl.kernel,
    mesh=_mesh,
    out_type=jax.ShapeDtypeStruct((SEQ, DIM // 8, NBT, 8, LANES), jnp.float32),
    scratch_types=[
        pltpu.VMEM((2, BT_PER_W, LANES), jnp.int32),
        pltpu.VMEM((2, BT_PER_W, LANES, DIM), jnp.float32),
        pltpu.VMEM((2, DIM // 8, BT_PER_W, 8, LANES), jnp.float32),
        pltpu.VMEM((16, 16), jnp.float32),
        pltpu.SemaphoreType.DMA,
        pltpu.SemaphoreType.DMA,
        pltpu.SemaphoreType.DMA,
        pltpu.SemaphoreType.DMA,
        pltpu.SemaphoreType.DMA,
        pltpu.SemaphoreType.DMA,
    ],
    compiler_params=pltpu.CompilerParams(use_tc_tiling_on_sc=False,
                                        needs_layout_passes=False),
)
def _gather_kernel(table_hbm, idx_hbm, out_hbm, idx_v, rows_v, tile_v, dbuf,
                   isem0, isem1, gsem0, gsem1, ssem0, ssem1):
    wid = lax.axis_index("s") * NC + lax.axis_index("c")
    bt0 = wid * BT_PER_W
    isem, gsem, ssem = (isem0, isem1), (gsem0, gsem1), (ssem0, ssem1)
    iota16 = lax.iota(jnp.int32, 16)

    def start_idx(j, b):
        pltpu.async_copy(idx_hbm.at[j, pl.ds(bt0, BT_PER_W)], idx_v.at[b],
                         isem[b])

    def wait_idx(b):
        pltpu.make_async_copy(idx_hbm.at[0, pl.ds(0, BT_PER_W)], idx_v.at[b],
                              isem[b]).wait()

    def start_gathers(b):
        for t in range(BT_PER_W):
            pltpu.async_copy(table_hbm.at[idx_v.at[b, t]], rows_v.at[b, t],
                             gsem[b])

    def wait_gathers(b):
        for t in range(BT_PER_W):
            pltpu.make_async_copy(table_hbm.at[pl.ds(0, LANES)],
                                  rows_v.at[b, t], gsem[b]).wait()

    def start_stores(j, b):
        for dt in range(DIM // 8):
            pltpu.async_copy(tile_v.at[b, dt],
                             out_hbm.at[j, dt, pl.ds(bt0, BT_PER_W)],
                             ssem[b])

    def wait_stores(b):
        for dt in range(DIM // 8):
            pltpu.make_async_copy(out_hbm.at[0, 0, pl.ds(0, BT_PER_W)],
                                  tile_v.at[b, dt], ssem[b]).wait()

    cols_k = [[dh * 16 + ((k + iota16) & 15) for k in range(16)]
              for dh in range(DIM // 16)]
    rows_r = [(16 + r - iota16) & 15 for r in range(16)]

    def transpose(b):
        def tr_body(lb, carry):
            row_ids = lb * 16 + iota16
            for t in range(BT_PER_W):
                src = rows_v.at[b, t]
                for dh in range(DIM // 16):
                    for k in range(16):
                        dbuf[k, pl.ds(0, 16)] = plsc.load_gather(
                            src, [row_ids, cols_k[dh][k]])
                    for r in range(16):
                        d = dh * 16 + r
                        tile_v[b, d // 8, t, d % 8, pl.ds(lb * 16, 16)] = (
                            plsc.load_gather(dbuf, [rows_r[r], iota16]))
            return carry

        lax.fori_loop(0, LANES // 16, tr_body, 0)

    pltpu.sync_copy(idx_hbm.at[0, pl.ds(bt0, BT_PER_W)], idx_v.at[0])
    start_gathers(0)
    start_idx(1, 1)

    def pair_body(i, carry):
        j0 = 2 * i
        not_first = i > 0
        not_last = i < SEQ // 2 - 1

        wait_idx(1)
        start_gathers(1)
        wait_gathers(0)
        pl.when(not_last)(lambda: start_idx(j0 + 2, 0))
        pl.when(not_first)(lambda: wait_stores(0))
        transpose(0)
        start_stores(j0, 0)

        @pl.when(not_last)
        def _():
            wait_idx(0)
            start_gathers(0)

        wait_gathers(1)
        pl.when(not_last)(lambda: start_idx(j0 + 3, 1))
        pl.when(not_first)(lambda: wait_stores(1))
        transpose(1)
        start_stores(j0 + 1, 1)
        return carry

    lax.fori_loop(0, SEQ // 2, pair_body, 0)

    wait_stores(0)
    wait_stores(1)


def kernel(idx, emb_weight):
    idx3 = jnp.transpose(idx).astype(jnp.int32).reshape(SEQ, NBT, LANES)
    out5 = _gather_kernel(emb_weight, idx3)
    return out5.transpose(2, 4, 0, 1, 3).reshape(BATCH, SEQ, DIM)


# speedup vs baseline: 1.0003x; 1.0003x over previous
import functools

import jax
import jax.numpy as jnp
from jax import lax
from jax.experimental import pallas as pl
from jax.experimental.pallas import tpu as pltpu
from jax.experimental.pallas import tpu_sc as plsc

BATCH = 16384
SEQ = 200
DIM = 32
LANES = 128
NBT = BATCH // LANES
NC, NS = 2, 16
NW = NC * NS
BT_PER_W = NBT // NW

_mesh = plsc.VectorSubcoreMesh(core_axis_name="c", subcore_axis_name="s")


@functools.partial(
    pl.kernel,
    mesh=_mesh,
    out_type=jax.ShapeDtypeStruct((SEQ, DIM // 8, NBT, 8, LANES), jnp.float32),
    scratch_types=[
        pltpu.VMEM((2, BT_PER_W, LANES), jnp.int32),
        pltpu.VMEM((2, BT_PER_W, LANES, DIM), jnp.float32),
        pltpu.VMEM((2, DIM // 8, BT_PER_W, 8, LANES), jnp.float32),
        pltpu.VMEM((2, 16, 16), jnp.float32),
        pltpu.SemaphoreType.DMA,
        pltpu.SemaphoreType.DMA,
        pltpu.SemaphoreType.DMA,
        pltpu.SemaphoreType.DMA,
        pltpu.SemaphoreType.DMA,
        pltpu.SemaphoreType.DMA,
    ],
    compiler_params=pltpu.CompilerParams(use_tc_tiling_on_sc=False,
                                        needs_layout_passes=False),
)
def _gather_kernel(table_hbm, idx_hbm, out_hbm, idx_v, rows_v, tile_v, dbuf,
                   isem0, isem1, gsem0, gsem1, ssem0, ssem1):
    wid = lax.axis_index("s") * NC + lax.axis_index("c")
    bt0 = wid * BT_PER_W
    isem, gsem, ssem = (isem0, isem1), (gsem0, gsem1), (ssem0, ssem1)
    iota16 = lax.iota(jnp.int32, 16)

    def start_idx(j, b):
        pltpu.async_copy(idx_hbm.at[j, pl.ds(bt0, BT_PER_W)], idx_v.at[b],
                         isem[b])

    def wait_idx(b):
        pltpu.make_async_copy(idx_hbm.at[0, pl.ds(0, BT_PER_W)], idx_v.at[b],
                              isem[b]).wait()

    def start_gathers(b):
        for t in range(BT_PER_W):
            pltpu.async_copy(table_hbm.at[idx_v.at[b, t]], rows_v.at[b, t],
                             gsem[b])

    def wait_gathers(b):
        for t in range(BT_PER_W):
            pltpu.make_async_copy(table_hbm.at[pl.ds(0, LANES)],
                                  rows_v.at[b, t], gsem[b]).wait()

    def start_stores(j, b):
        for dt in range(DIM // 8):
            pltpu.async_copy(tile_v.at[b, dt],
                             out_hbm.at[j, dt, pl.ds(bt0, BT_PER_W)],
                             ssem[b])

    def wait_stores(b):
        for dt in range(DIM // 8):
            pltpu.make_async_copy(out_hbm.at[0, 0, pl.ds(0, BT_PER_W)],
                                  tile_v.at[b, dt], ssem[b]).wait()

    cols_k = [[dh * 16 + ((k + iota16) & 15) for k in range(16)]
              for dh in range(DIM // 16)]
    rows_r = [(16 + r - iota16) & 15 for r in range(16)]

    def transpose(b):
        def tr_body(lb, carry):
            row_ids = lb * 16 + iota16
            for t in range(BT_PER_W):
                src = rows_v.at[b, t]
                for dh in range(DIM // 16):
                    db = dbuf.at[dh]
                    for k in range(16):
                        db[k, pl.ds(0, 16)] = plsc.load_gather(
                            src, [row_ids, cols_k[dh][k]])
                    for r in range(16):
                        d = dh * 16 + r
                        tile_v[b, d // 8, t, d % 8, pl.ds(lb * 16, 16)] = (
                            plsc.load_gather(db, [rows_r[r], iota16]))
            return carry

        lax.fori_loop(0, LANES // 16, tr_body, 0)

    pltpu.sync_copy(idx_hbm.at[0, pl.ds(bt0, BT_PER_W)], idx_v.at[0])
    start_gathers(0)
    start_idx(1, 1)

    def pair_body(i, carry):
        j0 = 2 * i
        not_first = i > 0
        not_last = i < SEQ // 2 - 1

        wait_idx(1)
        start_gathers(1)
        wait_gathers(0)
        pl.when(not_last)(lambda: start_idx(j0 + 2, 0))
        pl.when(not_first)(lambda: wait_stores(0))
        transpose(0)
        start_stores(j0, 0)

        @pl.when(not_last)
        def _():
            wait_idx(0)
            start_gathers(0)

        wait_gathers(1)
        pl.when(not_last)(lambda: start_idx(j0 + 3, 1))
        pl.when(not_first)(lambda: wait_stores(1))
        transpose(1)
        start_stores(j0 + 1, 1)
        return carry

    lax.fori_loop(0, SEQ // 2, pair_body, 0)

    wait_stores(0)
    wait_stores(1)


def kernel(idx, emb_weight):
    idx3 = jnp.transpose(idx).astype(jnp.int32).reshape(SEQ, NBT, LANES)
    out5 = _gather_kernel(emb_weight, idx3)
    return out5.transpose(2, 4, 0, 1, 3).reshape(BATCH, SEQ, DIM)
